# baseline (speedup 1.0000x reference)
import functools

import jax
import jax.numpy as jnp
from jax import lax
from jax.experimental import pallas as pl
from jax.experimental.pallas import tpu as pltpu
from jax.experimental.pallas import tpu_sc as plsc

G = 64
NC = 2
NS = 16
NW = NC * NS
K = 128

_mesh = plsc.VectorSubcoreMesh(core_axis_name="c", subcore_axis_name="s")


def _deg_body(dst_hbm, out_hbm, hist_v, didx_v, n, ept):
    c = lax.axis_index("c")
    s = lax.axis_index("s")
    wid = s * NC + c

    def zero(i, carry):
        hist_v[pl.ds(i * 16, 16)] = jnp.zeros((16,), jnp.float32)
        return carry

    lax.fori_loop(0, n // 16, zero, 0)
    pltpu.sync_copy(dst_hbm.at[pl.ds(wid * ept, ept)], didx_v)
    ones = jnp.ones((16,), jnp.float32)

    def upd(i, carry):
        idx = didx_v[pl.ds(i * 16, 16)]
        plsc.addupdate_scatter(hist_v, [idx], ones)
        return carry

    lax.fori_loop(0, ept // 16, upd, 0)
    pltpu.sync_copy(hist_v, out_hbm.at[wid])


def _make_deg_kernel(n, e):
    ept = e // NW
    return pl.kernel(
        functools.partial(_deg_body, n=n, ept=ept),
        out_type=jax.ShapeDtypeStruct((NW, n), jnp.float32),
        mesh=_mesh,
        scratch_types=[
            pltpu.VMEM((n,), jnp.float32),
            pltpu.VMEM((ept,), jnp.int32),
        ],
        compiler_params=pltpu.CompilerParams(needs_layout_passes=False),
    )


def _seg_body(hp_hbm, src_hbm, dst_hbm, out_hbm,
              sidx_v, draw0_v, draw1_v, draw2_v,
              didx0_v, didx1_v, didx2_v,
              rows0_v, rows1_v, rows2_v,
              drawe_v, didxe_v, rowse_v, stage_v, acc_sh,
              semd0, semd1, semd2,
              semg0, semg1, semg2,
              sems0, sems1, sems2, n, h, e):
    c = lax.axis_index("c")
    s = lax.axis_index("s")
    ept = e // NS
    nch = ept // K
    hn = n // NC
    rpt = (hn // NS) // 8 * 8
    tail = hn - rpt * NS

    srows = rpt // 3

    def zrow(i, carry):
        def zcol(j, inner):
            stage_v[i, pl.ds(j * 16, 16)] = jnp.zeros((16,), jnp.float32)
            return inner
        return lax.fori_loop(0, h // 16, zcol, carry)

    lax.fori_loop(0, srows, zrow, 0)
    for q in range(3):
        pltpu.sync_copy(stage_v, acc_sh.at[pl.ds(s * rpt + q * srows, srows)])
    if tail:
        @pl.when(s == NS - 1)
        def _():
            pltpu.sync_copy(stage_v.at[pl.ds(0, tail + 8)],
                            acc_sh.at[pl.ds(NS * rpt, tail + 8)])
    plsc.subcore_barrier()

    pltpu.sync_copy(src_hbm.at[pl.ds(s * ept, ept)], sidx_v)
    base = c * hn

    def gidx(k):
        return sidx_v.at[pl.ds(k * K, K)]

    def dst_off(k):
        return s * ept + k * K

    trash = hn + (lax.iota(jnp.int32, 16) & 7)

    def transform(draw_ref, didx_ref):
        for j in range(K // 16):
            d = draw_ref[pl.ds(j * 16, 16)] - base
            ok = (d >= 0) & (d < hn)
            didx_ref[pl.ds(j * 16, 16)] = jnp.where(ok, d, trash)

    draws = (draw0_v, draw1_v, draw2_v)
    didxs = (didx0_v, didx1_v, didx2_v)
    rowss = (rows0_v, rows1_v, rows2_v)
    semds = (semd0, semd1, semd2)
    semgs = (semg0, semg1, semg2)
    semss = (sems0, sems1, sems2)

    def wait_like(src, dst, sem):
        pltpu.make_async_copy(src, dst, sem).wait()

    for p in range(3):
        pltpu.async_copy(dst_hbm.at[pl.ds(dst_off(p), K)], draws[p], semds[p])
    for p in range(3):
        wait_like(dst_hbm.at[pl.ds(0, K)], draws[p], semds[p])
        transform(draws[p], didxs[p])
        pltpu.async_copy(dst_hbm.at[pl.ds(dst_off(p + 3), K)], draws[p],
                         semds[p])
        pltpu.async_copy(hp_hbm.at[gidx(p)], rowss[p], semgs[p])
    for p in range(3):
        wait_like(hp_hbm.at[pl.ds(0, K)], rowss[p], semgs[p])
        pltpu.sync_copy(rowss[p], acc_sh.at[didxs[p]], add=True)

    def triple(i, carry):
        for p in range(3):
            k = 3 * i + p
            wait_like(dst_hbm.at[pl.ds(0, K)], draws[p], semds[p])
            transform(draws[p], didxs[p])

            @pl.when(k + 3 < nch)
            def _(p=p, k=k):
                pltpu.async_copy(dst_hbm.at[pl.ds(dst_off(k + 3), K)],
                                 draws[p], semds[p])

            pltpu.async_copy(hp_hbm.at[gidx(k)], rowss[p], semgs[p])

        for p in range(3):
            wait_like(hp_hbm.at[pl.ds(0, K)], rowss[p], semgs[p])
            pltpu.sync_copy(rowss[p], acc_sh.at[didxs[p]], add=True)
        return carry

    lax.fori_loop(1, nch // 3, triple, 0)


    rem = ept - nch * K
    if rem:
        pltpu.sync_copy(dst_hbm.at[pl.ds(dst_off(nch), rem)], drawe_v)
        for j in range(rem // 16):
            d = drawe_v[pl.ds(j * 16, 16)] - base
            ok = (d >= 0) & (d < hn)
            didxe_v[pl.ds(j * 16, 16)] = jnp.where(ok, d, trash)
        pltpu.async_copy(hp_hbm.at[sidx_v.at[pl.ds(nch * K, rem)]], rowse_v,
                         semg0).wait()
        pltpu.sync_copy(rowse_v, acc_sh.at[didxe_v], add=True)
    plsc.subcore_barrier()

    for q in range(3):
        pltpu.sync_copy(acc_sh.at[pl.ds(s * rpt + q * srows, srows)], stage_v)
        pltpu.sync_copy(stage_v,
                        out_hbm.at[pl.ds(base + s * rpt + q * srows, srows)])
    if tail:
        @pl.when(s == NS - 1)
        def _():
            pltpu.sync_copy(acc_sh.at[pl.ds(NS * rpt, tail)],
                            stage_v.at[pl.ds(0, tail)])
            pltpu.sync_copy(stage_v.at[pl.ds(0, tail)],
                            out_hbm.at[pl.ds(base + NS * rpt, tail)])


def _make_seg_kernel(n, h, e):
    hn = n // NC
    rpt = (hn // NS) // 8 * 8
    return pl.kernel(
        functools.partial(_seg_body, n=n, h=h, e=e),
        out_type=jax.ShapeDtypeStruct((n, h), jnp.float32),
        mesh=_mesh,
        scratch_types=[
            pltpu.VMEM((e // NS,), jnp.int32),
            pltpu.VMEM((K,), jnp.int32),
            pltpu.VMEM((K,), jnp.int32),
            pltpu.VMEM((K,), jnp.int32),
            pltpu.VMEM((K,), jnp.int32),
            pltpu.VMEM((K,), jnp.int32),
            pltpu.VMEM((K,), jnp.int32),
            pltpu.VMEM((K, h), jnp.float32),
            pltpu.VMEM((K, h), jnp.float32),
            pltpu.VMEM((K, h), jnp.float32),
            pltpu.VMEM((((e // NS) % K) or 8,), jnp.int32),
            pltpu.VMEM((((e // NS) % K) or 8,), jnp.int32),
            pltpu.VMEM((((e // NS) % K) or 8, h), jnp.float32),
            pltpu.VMEM((rpt // 3, h), jnp.float32),
            pltpu.VMEM_SHARED((hn + 8, h), jnp.float32),
        ] + [pltpu.SemaphoreType.DMA] * 9,
        compiler_params=pltpu.CompilerParams(needs_layout_passes=False),
    )


def _tc_first(degp_ref, x_ref, w_ref, dis_ref, hp_ref):
    deg = jnp.sum(degp_ref[...], axis=0) + 1.0
    dis = lax.rsqrt(deg)
    dis_ref[...] = dis
    hp = jnp.dot(x_ref[...], w_ref[...], preferred_element_type=jnp.float32)
    hp_ref[...] = hp * dis[:, None]


def _tc_mid(sp_ref, hp_ref, dis_ref, b_ref, w_ref, out_ref):
    dis = dis_ref[...]
    t = sp_ref[...] + hp_ref[...]
    hcur = jnp.maximum(dis[:, None] * t + b_ref[...][None, :], 0.0)
    out = jnp.dot(hcur, w_ref[...], preferred_element_type=jnp.float32)
    out_ref[...] = out * dis[:, None]


def _tc_last(sp_ref, hp_ref, dis_ref, b_ref, batch_ref, wl_ref, bl_ref,
             out_ref):
    dis = dis_ref[...]
    t = sp_ref[...] + hp_ref[...]
    hcur = jnp.maximum(dis[:, None] * t + b_ref[...][None, :], 0.0)
    n = hcur.shape[0]
    gids = lax.broadcasted_iota(jnp.int32, (G, n), 0)
    onehot = (batch_ref[...][None, :] == gids).astype(jnp.float32)
    cnt = jnp.sum(onehot, axis=1)
    pooled = jnp.dot(onehot, hcur, preferred_element_type=jnp.float32)
    pooled = pooled / jnp.clip(cnt, 1.0)[:, None]
    out = jnp.dot(pooled, wl_ref[...], preferred_element_type=jnp.float32)
    out_ref[...] = out + bl_ref[...][None, :]


def kernel(x, edge_index, batch, W1, b1, W2, b2, W3, b3, Wl, bl):
    n, d = x.shape
    h = W1.shape[1]
    e = edge_index.shape[1]
    c_out = Wl.shape[1]

    src = edge_index[0]
    dst = edge_index[1]

    deg_kernel = _make_deg_kernel(n, e)
    seg_kernel = _make_seg_kernel(n, h, e)

    degp = deg_kernel(dst)

    tc_first = pl.pallas_call(
        _tc_first,
        out_shape=[jax.ShapeDtypeStruct((n,), jnp.float32),
                   jax.ShapeDtypeStruct((n, h), jnp.float32)],
    )
    dis, hp1 = tc_first(degp, x, W1)

    tc_mid = pl.pallas_call(
        _tc_mid,
        out_shape=jax.ShapeDtypeStruct((n, h), jnp.float32),
    )

    sp1 = seg_kernel(hp1, src, dst)
    hp2 = tc_mid(sp1, hp1, dis, b1, W2)
    sp2 = seg_kernel(hp2, src, dst)
    hp3 = tc_mid(sp2, hp2, dis, b2, W3)
    sp3 = seg_kernel(hp3, src, dst)

    tc_last = pl.pallas_call(
        _tc_last,
        out_shape=jax.ShapeDtypeStruct((G, c_out), jnp.float32),
    )
    return tc_last(sp3, hp3, dis, b3, batch, Wl, bl)


# speedup vs baseline: 1.4565x; 1.4565x over previous
import functools

import jax
import jax.numpy as jnp
from jax import lax
from jax.experimental import pallas as pl
from jax.experimental.pallas import tpu as pltpu
from jax.experimental.pallas import tpu_sc as plsc

G = 64
NC = 2
NS = 16
NW = NC * NS
K = 128

_mesh = plsc.VectorSubcoreMesh(core_axis_name="c", subcore_axis_name="s")


def _deg_body(dst_hbm, out_hbm, hist_v, didx_v, n, ept):
    c = lax.axis_index("c")
    s = lax.axis_index("s")
    wid = s * NC + c

    def zero(i, carry):
        hist_v[pl.ds(i * 16, 16)] = jnp.zeros((16,), jnp.float32)
        return carry

    lax.fori_loop(0, n // 16, zero, 0)
    pltpu.sync_copy(dst_hbm.at[pl.ds(wid * ept, ept)], didx_v)
    ones = jnp.ones((16,), jnp.float32)

    def upd(i, carry):
        idx = didx_v[pl.ds(i * 16, 16)]
        plsc.addupdate_scatter(hist_v, [idx], ones)
        return carry

    lax.fori_loop(0, ept // 16, upd, 0)
    pltpu.sync_copy(hist_v, out_hbm.at[wid])


def _make_deg_kernel(n, e):
    ept = e // NW
    return pl.kernel(
        functools.partial(_deg_body, n=n, ept=ept),
        out_type=jax.ShapeDtypeStruct((NW, n), jnp.float32),
        mesh=_mesh,
        scratch_types=[
            pltpu.VMEM((n,), jnp.float32),
            pltpu.VMEM((ept,), jnp.int32),
        ],
        compiler_params=pltpu.CompilerParams(needs_layout_passes=False),
    )


def _seg_body(hp_hbm, src_hbm, dst_hbm, out_hbm,
              sidx_v, draw0_v, draw1_v, draw2_v,
              didx0_v, didx1_v, didx2_v,
              rows0_v, rows1_v, rows2_v,
              drawe_v, didxe_v, rowse_v, stage_v, acc_sh,
              semd0, semd1, semd2,
              semg0, semg1, semg2, n, h, e):
    c = lax.axis_index("c")
    s = lax.axis_index("s")
    ept = e // NS
    nch = ept // K
    hn = n // NC
    rpt = (hn // NS) // 8 * 8
    tail = hn - rpt * NS

    srows = rpt // 3

    def zrow(i, carry):
        def zcol(j, inner):
            stage_v[i, pl.ds(j * 16, 16)] = jnp.zeros((16,), jnp.float32)
            return inner
        return lax.fori_loop(0, h // 16, zcol, carry)

    lax.fori_loop(0, srows, zrow, 0)
    for q in range(3):
        pltpu.sync_copy(stage_v, acc_sh.at[pl.ds(s * rpt + q * srows, srows)])
    if tail:
        @pl.when(s == NS - 1)
        def _():
            pltpu.sync_copy(stage_v.at[pl.ds(0, tail + 8)],
                            acc_sh.at[pl.ds(NS * rpt, tail + 8)])
    plsc.subcore_barrier()

    pltpu.sync_copy(src_hbm.at[pl.ds(s * ept, ept)], sidx_v)
    base = c * hn

    def gidx(k):
        return sidx_v.at[pl.ds(k * K, K)]

    def dst_off(k):
        return s * ept + k * K

    trash = hn + (lax.iota(jnp.int32, 16) & 7)

    def transform(draw_ref, didx_ref):
        for j in range(K // 16):
            d = draw_ref[pl.ds(j * 16, 16)] - base
            ok = (d >= 0) & (d < hn)
            didx_ref[pl.ds(j * 16, 16)] = jnp.where(ok, d, trash)

    draws = (draw0_v, draw1_v, draw2_v)
    didxs = (didx0_v, didx1_v, didx2_v)
    rowss = (rows0_v, rows1_v, rows2_v)
    semds = (semd0, semd1, semd2)
    semgs = (semg0, semg1, semg2)

    def wait_like(src, dst, sem):
        pltpu.make_async_copy(src, dst, sem).wait()

    for p in range(3):
        pltpu.async_copy(dst_hbm.at[pl.ds(dst_off(p), K)], draws[p], semds[p])
        pltpu.async_copy(hp_hbm.at[gidx(p)], rowss[p], semgs[p])

    def triple(i, carry):
        for p in range(3):
            k = 3 * i + p
            wait_like(dst_hbm.at[pl.ds(0, K)], draws[p], semds[p])
            transform(draws[p], didxs[p])

            @pl.when(k + 3 < nch)
            def _(p=p, k=k):
                pltpu.async_copy(dst_hbm.at[pl.ds(dst_off(k + 3), K)],
                                 draws[p], semds[p])

            wait_like(hp_hbm.at[pl.ds(0, K)], rowss[p], semgs[p])
            pltpu.sync_copy(rowss[p], acc_sh.at[didxs[p]], add=True)

            @pl.when(k + 3 < nch)
            def _(p=p, k=k):
                pltpu.async_copy(hp_hbm.at[gidx(k + 3)], rowss[p], semgs[p])

        return carry

    lax.fori_loop(0, nch // 3, triple, 0)


    rem = ept - nch * K
    if rem:
        pltpu.sync_copy(dst_hbm.at[pl.ds(dst_off(nch), rem)], drawe_v)
        for j in range(rem // 16):
            d = drawe_v[pl.ds(j * 16, 16)] - base
            ok = (d >= 0) & (d < hn)
            didxe_v[pl.ds(j * 16, 16)] = jnp.where(ok, d, trash)
        pltpu.async_copy(hp_hbm.at[sidx_v.at[pl.ds(nch * K, rem)]], rowse_v,
                         semg0).wait()
        pltpu.sync_copy(rowse_v, acc_sh.at[didxe_v], add=True)
    plsc.subcore_barrier()

    for q in range(3):
        pltpu.sync_copy(acc_sh.at[pl.ds(s * rpt + q * srows, srows)], stage_v)
        pltpu.sync_copy(stage_v,
                        out_hbm.at[pl.ds(base + s * rpt + q * srows, srows)])
    if tail:
        @pl.when(s == NS - 1)
        def _():
            pltpu.sync_copy(acc_sh.at[pl.ds(NS * rpt, tail)],
                            stage_v.at[pl.ds(0, tail)])
            pltpu.sync_copy(stage_v.at[pl.ds(0, tail)],
                            out_hbm.at[pl.ds(base + NS * rpt, tail)])


def _make_seg_kernel(n, h, e):
    hn = n // NC
    rpt = (hn // NS) // 8 * 8
    return pl.kernel(
        functools.partial(_seg_body, n=n, h=h, e=e),
        out_type=jax.ShapeDtypeStruct((n, h), jnp.float32),
        mesh=_mesh,
        scratch_types=[
            pltpu.VMEM((e // NS,), jnp.int32),
            pltpu.VMEM((K,), jnp.int32),
            pltpu.VMEM((K,), jnp.int32),
            pltpu.VMEM((K,), jnp.int32),
            pltpu.VMEM((K,), jnp.int32),
            pltpu.VMEM((K,), jnp.int32),
            pltpu.VMEM((K,), jnp.int32),
            pltpu.VMEM((K, h), jnp.float32),
            pltpu.VMEM((K, h), jnp.float32),
            pltpu.VMEM((K, h), jnp.float32),
            pltpu.VMEM((((e // NS) % K) or 8,), jnp.int32),
            pltpu.VMEM((((e // NS) % K) or 8,), jnp.int32),
            pltpu.VMEM((((e // NS) % K) or 8, h), jnp.float32),
            pltpu.VMEM((rpt // 3, h), jnp.float32),
            pltpu.VMEM_SHARED((hn + 8, h), jnp.float32),
        ] + [pltpu.SemaphoreType.DMA] * 6,
        compiler_params=pltpu.CompilerParams(needs_layout_passes=False),
    )


def _tc_first(degp_ref, x_ref, w_ref, dis_ref, hp_ref):
    deg = jnp.sum(degp_ref[...], axis=0) + 1.0
    dis = lax.rsqrt(deg)
    dis_ref[...] = dis
    hp = jnp.dot(x_ref[...], w_ref[...], preferred_element_type=jnp.float32)
    hp_ref[...] = hp * dis[:, None]


def _tc_mid(sp_ref, hp_ref, dis_ref, b_ref, w_ref, out_ref):
    dis = dis_ref[...]
    t = sp_ref[...] + hp_ref[...]
    hcur = jnp.maximum(dis[:, None] * t + b_ref[...][None, :], 0.0)
    out = jnp.dot(hcur, w_ref[...], preferred_element_type=jnp.float32)
    out_ref[...] = out * dis[:, None]


def _tc_last(sp_ref, hp_ref, dis_ref, b_ref, batch_ref, wl_ref, bl_ref,
             out_ref):
    dis = dis_ref[...]
    t = sp_ref[...] + hp_ref[...]
    hcur = jnp.maximum(dis[:, None] * t + b_ref[...][None, :], 0.0)
    n = hcur.shape[0]
    gids = lax.broadcasted_iota(jnp.int32, (G, n), 0)
    onehot = (batch_ref[...][None, :] == gids).astype(jnp.float32)
    cnt = jnp.sum(onehot, axis=1)
    pooled = jnp.dot(onehot, hcur, preferred_element_type=jnp.float32)
    pooled = pooled / jnp.clip(cnt, 1.0)[:, None]
    out = jnp.dot(pooled, wl_ref[...], preferred_element_type=jnp.float32)
    out_ref[...] = out + bl_ref[...][None, :]


def kernel(x, edge_index, batch, W1, b1, W2, b2, W3, b3, Wl, bl):
    n, d = x.shape
    h = W1.shape[1]
    e = edge_index.shape[1]
    c_out = Wl.shape[1]

    src = edge_index[0]
    dst = edge_index[1]

    deg_kernel = _make_deg_kernel(n, e)
    seg_kernel = _make_seg_kernel(n, h, e)

    degp = deg_kernel(dst)

    tc_first = pl.pallas_call(
        _tc_first,
        out_shape=[jax.ShapeDtypeStruct((n,), jnp.float32),
                   jax.ShapeDtypeStruct((n, h), jnp.float32)],
    )
    dis, hp1 = tc_first(degp, x, W1)

    tc_mid = pl.pallas_call(
        _tc_mid,
        out_shape=jax.ShapeDtypeStruct((n, h), jnp.float32),
    )

    sp1 = seg_kernel(hp1, src, dst)
    hp2 = tc_mid(sp1, hp1, dis, b1, W2)
    sp2 = seg_kernel(hp2, src, dst)
    hp3 = tc_mid(sp2, hp2, dis, b2, W3)
    sp3 = seg_kernel(hp3, src, dst)

    tc_last = pl.pallas_call(
        _tc_last,
        out_shape=jax.ShapeDtypeStruct((G, c_out), jnp.float32),
    )
    return tc_last(sp3, hp3, dis, b3, batch, Wl, bl)
